# Initial kernel scaffold; baseline (speedup 1.0000x reference)
import jax
import jax.numpy as jnp
from jax.experimental import pallas as pl


def kernel(x_nchw, w1, w2):
    raise NotImplementedError("write your pallas kernel here")


# speedup vs baseline: 1.0658x; 1.0658x over previous
import jax
import jax.numpy as jnp
from jax.experimental import pallas as pl
from jax.experimental.pallas import tpu as pltpu


def _epa_gate_kernel(x_ref, w1_ref, w2_ref, o_ref):
    x = x_ref[...]
    xb = x.astype(jnp.bfloat16)
    h = jnp.dot(w1_ref[...], xb, preferred_element_type=jnp.float32)
    h = jnp.maximum(h, 0.0).astype(jnp.bfloat16)
    s = jnp.dot(w2_ref[...], h, preferred_element_type=jnp.float32)
    o_ref[...] = x * jax.nn.sigmoid(s)


def _even_tile(hw, cap=2048):
    best = None
    t = 128
    while t <= min(hw, cap):
        if hw % t == 0:
            best = t
        t += 128
    return best if best is not None else hw


def kernel(x_nchw, w1, w2):
    N, C, H, W = x_nchw.shape
    HW = H * W

    x3 = x_nchw.reshape(N, C, HW)
    w1m = w1[:, :, 0, 0].astype(jnp.bfloat16)
    w2m = w2[:, :, 0, 0].astype(jnp.bfloat16)

    thw = _even_tile(HW)
    n_hw_tiles = HW // thw

    out3 = pl.pallas_call(
        _epa_gate_kernel,
        out_shape=jax.ShapeDtypeStruct((N, C, HW), x_nchw.dtype),
        grid=(N, n_hw_tiles),
        in_specs=[
            pl.BlockSpec((None, C, thw), lambda n, p: (n, 0, p)),
            pl.BlockSpec((C, C), lambda n, p: (0, 0)),
            pl.BlockSpec((C, C), lambda n, p: (0, 0)),
        ],
        out_specs=pl.BlockSpec((None, C, thw), lambda n, p: (n, 0, p)),
        compiler_params=pltpu.CompilerParams(
            dimension_semantics=("parallel", "parallel")),
    )(x3, w1m, w2m)

    return out3.reshape(N, C, H, W)


# speedup vs baseline: 1.1281x; 1.0584x over previous
import jax
import jax.numpy as jnp
from jax.experimental import pallas as pl
from jax.experimental.pallas import tpu as pltpu


def _epa_gate_kernel(x_ref, w1_ref, w2_ref, o_ref):
    x = x_ref[...]
    xb = x.astype(jnp.bfloat16)
    h = jnp.dot(w1_ref[...], xb, preferred_element_type=jnp.float32)
    h = jnp.maximum(h, 0.0).astype(jnp.bfloat16)
    s = jnp.dot(w2_ref[...], h, preferred_element_type=jnp.float32)
    o_ref[...] = x * jax.nn.sigmoid(s)


def _even_tile(hw, cap=2048):
    best = None
    t = 128
    while t <= min(hw, cap):
        if hw % t == 0:
            best = t
        t += 128
    return best if best is not None else hw


def kernel(x_nchw, w1, w2):
    N, C, H, W = x_nchw.shape
    HW = H * W

    x3 = x_nchw.reshape(N, C, HW)
    w1m = w1[:, :, 0, 0].astype(jnp.bfloat16)
    w2m = w2[:, :, 0, 0].astype(jnp.bfloat16)

    thw = _even_tile(HW, cap=4096)
    n_hw_tiles = HW // thw

    out3 = pl.pallas_call(
        _epa_gate_kernel,
        out_shape=jax.ShapeDtypeStruct((N, C, HW), x_nchw.dtype),
        grid=(N, n_hw_tiles),
        in_specs=[
            pl.BlockSpec((None, C, thw), lambda n, p: (n, 0, p)),
            pl.BlockSpec((C, C), lambda n, p: (0, 0)),
            pl.BlockSpec((C, C), lambda n, p: (0, 0)),
        ],
        out_specs=pl.BlockSpec((None, C, thw), lambda n, p: (n, 0, p)),
        compiler_params=pltpu.CompilerParams(
            dimension_semantics=("parallel", "parallel")),
    )(x3, w1m, w2m)

    return out3.reshape(N, C, H, W)
